# baseline (speedup 1.0000x reference)
import functools

import jax
import jax.numpy as jnp
from jax import lax
from jax.experimental import pallas as pl
from jax.experimental.pallas import tpu as pltpu
from jax.experimental.pallas import tpu_sc as plsc

_NC = 2
_NS = 16
_NW = _NC * _NS

_CH = 128
_LANES = 16
_VC = 512


def _diag_vectors():
    iota = lax.iota(jnp.int32, _LANES)
    rot = [jnp.bitwise_and(iota + j, _LANES - 1) for j in range(_LANES)]
    return iota, rot


def _format_table(table):
    V, D = table.shape
    tT = table.T
    n_full = V // _VC
    per_w = n_full // _NW
    rem = n_full - per_w * _NW

    mesh = plsc.VectorSubcoreMesh(core_axis_name="c", subcore_axis_name="s")

    @functools.partial(
        pl.kernel,
        out_type=jax.ShapeDtypeStruct((V * D,), table.dtype),
        mesh=mesh,
        scratch_types=[
            pltpu.VMEM((D, _VC), jnp.float32),
            pltpu.VMEM((D, _VC), jnp.float32),
            pltpu.VMEM((_VC * D,), jnp.float32),
            pltpu.SemaphoreType.DMA,
            pltpu.SemaphoreType.DMA,
            pltpu.SemaphoreType.DMA,
        ],
        compiler_params=pltpu.CompilerParams(needs_layout_passes=False),
    )
    def run(tT_hbm, out_hbm, buf0, buf1, cbuf, si0, si1, so):
        wid = lax.axis_index("s") * _NC + lax.axis_index("c")
        bufs = (buf0, buf1)
        sis = (si0, si1)
        iota, rot = _diag_vectors()
        sb = [iota * D + rot[j] for j in range(_LANES)]

        def chunk_id(i):
            return wid + i * _NW

        def fire_in(i, b, width=_VC):
            v0 = chunk_id(i) * _VC
            pltpu.async_copy(
                tT_hbm.at[:, pl.ds(v0, width)],
                bufs[b].at[:, pl.ds(0, width)],
                sis[b],
            )

        def wait_in(i, b, width=_VC):
            v0 = chunk_id(i) * _VC
            pltpu.make_async_copy(
                tT_hbm.at[:, pl.ds(v0, width)],
                bufs[b].at[:, pl.ds(0, width)],
                sis[b],
            ).wait()

        def out_pair(i, width=_VC):
            v0 = chunk_id(i) * _VC
            return (cbuf.at[pl.ds(0, width * D)],
                    out_hbm.at[pl.ds(v0 * D, width * D)])

        def fire_out(i, width=_VC):
            src, dst = out_pair(i, width)
            pltpu.async_copy(src, dst, so)

        def wait_out(i, width=_VC):
            src, dst = out_pair(i, width)
            pltpu.make_async_copy(src, dst, so).wait()

        span = (_LANES - 1) * D + _LANES

        def transpose(b, n_vblk):
            rbuf = bufs[b]
            for fi in range(D // _LANES):
                f0 = fi * _LANES
                fvecs = [rot[j] + f0 for j in range(_LANES)]

                def blk(kv, carry):
                    v0 = kv * _LANES
                    vvec = iota + v0
                    off = v0 * D + f0
                    for j in range(_LANES):
                        vals = plsc.load_gather(rbuf, [fvecs[j], vvec])
                        plsc.store_scatter(cbuf, [sb[j] + off], vals)
                    return carry

                lax.fori_loop(0, n_vblk, blk, 0)

        nv = _VC // _LANES

        fire_in(0, 0)
        fire_in(1, 1)
        wait_in(0, 0)
        transpose(0, nv)
        fire_out(0)
        fire_in(2, 0)

        def body(i, carry):
            c1 = 2 * i + 1
            wait_in(c1, 1)
            wait_out(c1 - 1)
            transpose(1, nv)
            fire_out(c1)
            fire_in(c1 + 2, 1)
            c2 = 2 * i + 2
            wait_in(c2, 0)
            wait_out(c2 - 1)
            transpose(0, nv)
            fire_out(c2)
            fire_in(c2 + 2, 0)
            return carry

        lax.fori_loop(0, (per_w - 3) // 2, body, 0)

        c = per_w - 2
        wait_in(c, 1)
        wait_out(c - 1)
        transpose(1, nv)
        fire_out(c)
        c = per_w - 1
        wait_in(c, 0)
        wait_out(c - 1)
        transpose(0, nv)
        fire_out(c)
        wait_out(c)

        @pl.when(wid == 0)
        def _():
            v0 = (n_full - 1) * _VC
            pltpu.sync_copy(tT_hbm.at[:, pl.ds(v0, _VC)], buf1)
            transpose(1, nv)
            pltpu.async_copy(cbuf, out_hbm.at[pl.ds(v0 * D, _VC * D)],
                             so).wait()


    return run(tT)


def _gather(ids3, table2, tail_rows, thr, H, B, D):
    n_bhi = B // _CH
    n_cells = H * n_bhi
    per_w = n_cells // _NW
    fhi = D // 8
    cell_elems = _CH * D

    mesh = plsc.VectorSubcoreMesh(core_axis_name="c", subcore_axis_name="s")

    @functools.partial(
        pl.kernel,
        out_type=jax.ShapeDtypeStruct((H, fhi, n_bhi, 8 * _CH), table2.dtype),
        mesh=mesh,
        scratch_types=[
            pltpu.VMEM((per_w, _CH), jnp.int32),
            pltpu.VMEM(tail_rows.shape, jnp.float32),
            pltpu.VMEM((_CH, D), jnp.float32),
            pltpu.VMEM((_CH, D), jnp.float32),
            pltpu.VMEM((cell_elems,), jnp.float32),
            pltpu.VMEM((cell_elems,), jnp.float32),
            pltpu.SemaphoreType.DMA,
            pltpu.SemaphoreType.DMA,
            pltpu.SemaphoreType.DMA,
            pltpu.SemaphoreType.DMA,
        ],
        compiler_params=pltpu.CompilerParams(
            use_tc_tiling_on_sc=False, needs_layout_passes=False
        ),
    )
    def run(ids_hbm, table_hbm, tail_hbm, out_hbm, idx_v, tail_v,
            rows0, rows1, cell0, cell1, sg0, sg1, so0, so1):
        wid = lax.axis_index("s") * _NC + lax.axis_index("c")
        base = wid * per_w
        pltpu.sync_copy(ids_hbm.at[pl.ds(base, per_w)], idx_v)
        pltpu.sync_copy(tail_hbm, tail_v)

        rows = (rows0, rows1)
        cells = (cell0, cell1)
        sgs = (sg0, sg1)
        sos = (so0, so1)

        iota, rot = _diag_vectors()
        sb = [rot[j] * _CH + iota for j in range(_LANES)]

        def fire_gather(l, b):
            pltpu.async_copy(table_hbm.at[idx_v.at[l]], rows[b], sgs[b])

        def wait_gather(l, b):
            pltpu.make_async_copy(table_hbm.at[idx_v.at[l]], rows[b],
                                  sgs[b]).wait()

        def out_slices(l, b):
            c = base + l
            h = c // n_bhi
            bhi = c % n_bhi
            return [(cells[b].at[pl.ds(q * 8 * _CH, 8 * _CH)],
                     out_hbm.at[h, q, bhi]) for q in range(fhi)]

        def fire_out(l, b):
            for src, dst in out_slices(l, b):
                pltpu.async_copy(src, dst, sos[b])

        def wait_out(l, b):
            for src, dst in out_slices(l, b):
                pltpu.make_async_copy(src, dst, sos[b]).wait()

        n_blk_b = _CH // _LANES

        def fix_tail(l, b):
            ivs = [idx_v[l, pl.ds(bc * _LANES, _LANES)]
                   for bc in range(n_blk_b)]
            mx = jnp.max(ivs[0])
            for bc in range(1, n_blk_b):
                mx = jnp.maximum(mx, jnp.max(ivs[bc]))

            @pl.when(mx >= thr)
            def _():
                for bc in range(n_blk_b):
                    iv = ivs[bc]
                    mask = iv >= thr
                    idxc = jnp.maximum(iv - thr, 0)
                    bvec = iota + bc * _LANES

                    def fbody(f, carry):
                        fv = jnp.full((_LANES,), f, jnp.int32)
                        vals = plsc.load_gather(tail_v, [idxc, fv])
                        plsc.store_scatter(rows[b], [bvec, fv], vals,
                                           mask=mask)
                        return carry

                    lax.fori_loop(0, D, fbody, 0)

        def transpose(b):
            rbuf = rows[b]
            cbuf = cells[b]
            for fi in range(D // _LANES):
                f0 = fi * _LANES
                fvecs = [rot[j] + f0 for j in range(_LANES)]

                def blk(kb, carry):
                    b0 = kb * _LANES
                    bvec = iota + b0
                    off = f0 * _CH + b0
                    for j in range(_LANES):
                        vals = plsc.load_gather(rbuf, [bvec, fvecs[j]])
                        plsc.store_scatter(cbuf, [sb[j] + off], vals)
                    return carry

                lax.fori_loop(0, n_blk_b, blk, 0)

        fire_gather(0, 0)
        fire_gather(1, 1)
        wait_gather(0, 0)
        fix_tail(0, 0)
        transpose(0)
        fire_out(0, 0)

        def body(i, carry):
            l1 = 2 * i + 1
            wait_gather(l1, 1)
            fire_gather(l1 + 1, 0)
            fix_tail(l1, 1)
            transpose(1)
            wait_out(l1 - 1, 0)
            fire_out(l1, 1)
            l2 = 2 * i + 2
            wait_gather(l2, 0)
            fire_gather(l2 + 1, 1)
            fix_tail(l2, 0)
            transpose(0)
            wait_out(l2 - 1, 1)
            fire_out(l2, 0)
            return carry

        lax.fori_loop(0, (per_w - 2) // 2, body, 0)

        l_last = per_w - 1
        wait_gather(l_last, 1)
        fix_tail(l_last, 1)
        transpose(1)
        wait_out(l_last - 1, 0)
        fire_out(l_last, 1)
        wait_out(l_last, 1)

    return run(ids3, table2, tail_rows)


def kernel(input_ids, table):
    B, H = input_ids.shape
    V, D = table.shape

    t_lin = _format_table(table)
    table2 = t_lin.reshape(V, D)
    thr = (V // _VC) * _VC
    tail_rows = table[thr:, :]

    ids3 = input_ids.T.reshape(H * (B // _CH), _CH)

    out5 = _gather(ids3, table2, tail_rows, thr, H, B, D)
    out5 = out5.reshape(H, D // 8, B // _CH, 8, _CH)
    return out5.transpose(2, 4, 0, 1, 3).reshape(B, H, D)


# speedup vs baseline: 1.0817x; 1.0817x over previous
import functools

import jax
import jax.numpy as jnp
from jax import lax
from jax.experimental import pallas as pl
from jax.experimental.pallas import tpu as pltpu
from jax.experimental.pallas import tpu_sc as plsc

_NC = 2
_NS = 16
_NW = _NC * _NS

_CH = 128
_LANES = 16


def kernel(input_ids, table):
    B, H = input_ids.shape
    V, D = table.shape
    n_bhi = B // _CH
    n_cells = H * n_bhi
    per_w = n_cells // _NW
    fhi = D // 8
    cell_elems = _CH * D

    ids2 = input_ids.T.reshape(n_cells, _CH)

    mesh = plsc.VectorSubcoreMesh(core_axis_name="c", subcore_axis_name="s")

    @functools.partial(
        pl.kernel,
        out_type=jax.ShapeDtypeStruct((H, fhi, n_bhi, 8 * _CH), table.dtype),
        mesh=mesh,
        scratch_types=[
            pltpu.VMEM((per_w, _CH), jnp.int32),
            pltpu.VMEM((_CH, D), jnp.float32),
            pltpu.VMEM((_CH, D), jnp.float32),
            pltpu.VMEM((cell_elems,), jnp.float32),
            pltpu.VMEM((cell_elems,), jnp.float32),
            pltpu.SemaphoreType.DMA,
            pltpu.SemaphoreType.DMA,
            pltpu.SemaphoreType.DMA,
            pltpu.SemaphoreType.DMA,
        ],
        compiler_params=pltpu.CompilerParams(
            use_tc_tiling_on_sc=False, needs_layout_passes=False
        ),
    )
    def run(ids_hbm, table_hbm, out_hbm, idx_v, rows0, rows1, cell0, cell1,
            sg0, sg1, so0, so1):
        wid = lax.axis_index("s") * _NC + lax.axis_index("c")
        base = wid * per_w
        pltpu.sync_copy(ids_hbm.at[pl.ds(base, per_w)], idx_v)

        rows = (rows0, rows1)
        cells = (cell0, cell1)
        sgs = (sg0, sg1)
        sos = (so0, so1)

        iota = lax.iota(jnp.int32, _LANES)
        rot = [jnp.bitwise_and(iota + j, _LANES - 1) for j in range(_LANES)]
        sbase = [rot[j] * _CH + iota for j in range(_LANES)]

        def fire_gather(l, b):
            pltpu.async_copy(table_hbm.at[idx_v.at[l]], rows[b], sgs[b])

        def wait_gather(l, b):
            pltpu.make_async_copy(table_hbm.at[idx_v.at[l]], rows[b],
                                  sgs[b]).wait()

        def out_slices(l, b):
            c = base + l
            h = c // n_bhi
            bhi = c % n_bhi
            return [
                (cells[b].at[pl.ds(q * 8 * _CH, 8 * _CH)],
                 out_hbm.at[h, q, bhi])
                for q in range(fhi)
            ]

        def fire_out(l, b):
            for src, dst in out_slices(l, b):
                pltpu.async_copy(src, dst, sos[b])

        def wait_out(l, b):
            for src, dst in out_slices(l, b):
                pltpu.make_async_copy(src, dst, sos[b]).wait()

        n_blk_b = _CH // _LANES

        def transpose(b):
            rbuf = rows[b]
            cbuf = cells[b]

            def blk(k, carry):
                f0 = (k // n_blk_b) * _LANES
                b0 = (k % n_blk_b) * _LANES
                bvec = iota + b0
                off = f0 * _CH + b0
                for j in range(_LANES):
                    vals = plsc.load_gather(rbuf, [bvec, rot[j] + f0])
                    plsc.store_scatter(cbuf, [sbase[j] + off], vals)
                return carry

            lax.fori_loop(0, (D // _LANES) * n_blk_b, blk, 0)

        fire_gather(0, 0)
        fire_gather(1, 1)
        wait_gather(0, 0)
        transpose(0)
        fire_out(0, 0)

        def body(i, carry):
            l1 = 2 * i + 1
            wait_gather(l1, 1)
            fire_gather(l1 + 1, 0)
            transpose(1)
            wait_out(l1 - 1, 0)
            fire_out(l1, 1)
            l2 = 2 * i + 2
            wait_gather(l2, 0)
            fire_gather(l2 + 1, 1)
            transpose(0)
            wait_out(l2 - 1, 1)
            fire_out(l2, 0)
            return carry

        lax.fori_loop(0, (per_w - 2) // 2, body, 0)

        l_last = per_w - 1
        wait_gather(l_last, 1)
        transpose(1)
        wait_out(l_last - 1, 0)
        fire_out(l_last, 1)
        wait_out(l_last, 1)

    out5 = run(ids2, table)
    out5 = out5.reshape(H, fhi, n_bhi, 8, _CH)
    return out5.transpose(2, 4, 0, 1, 3).reshape(B, H, D)
